# baseline (speedup 1.0000x reference)
import functools

import jax
import jax.numpy as jnp
from jax import lax
from jax.experimental import pallas as pl
from jax.experimental.pallas import tpu as pltpu
from jax.experimental.pallas import tpu_sc as plsc

BATCH = 16384
EMBED = 64
PAIR = 2 * EMBED
CHUNK = 128


@functools.lru_cache(maxsize=None)
def _make_gather(num_pairs_u, num_pairs_i):
    info = plsc.get_sparse_core_info()
    nc, ns = info.num_cores, info.num_subcores
    nw = nc * ns
    bpw = BATCH // nw
    half = bpw // 2
    nch = half // CHUNK

    mesh = plsc.VectorSubcoreMesh(core_axis_name="c", subcore_axis_name="s")

    @functools.partial(
        pl.kernel,
        mesh=mesh,
        out_type=[
            jax.ShapeDtypeStruct((BATCH, PAIR), jnp.float32),
            jax.ShapeDtypeStruct((BATCH, PAIR), jnp.float32),
        ],
        scratch_types=[
            pltpu.VMEM((2 * nch, CHUNK), jnp.int32),
            pltpu.VMEM((2 * nch, CHUNK), jnp.int32),
            pltpu.VMEM((half, PAIR), jnp.float32),
            pltpu.VMEM((half, PAIR), jnp.float32),
            pltpu.SemaphoreType.DMA,
        ],
    )
    def gather_k(uid_hbm, iid_hbm, utab_hbm, itab_hbm, uout_hbm, iout_hbm,
                 uidx, iidx, upairs, ipairs, sem):
        wid = lax.axis_index("s") * nc + lax.axis_index("c")
        base = wid * bpw
        pltpu.sync_copy(uid_hbm.at[wid], uidx)
        pltpu.sync_copy(iid_hbm.at[wid], iidx)
        for p in range(2):
            copies = []
            for j in range(nch):
                copies.append(pltpu.async_copy(
                    utab_hbm.at[uidx.at[p * nch + j]],
                    upairs.at[pl.ds(j * CHUNK, CHUNK)], sem))
                copies.append(pltpu.async_copy(
                    itab_hbm.at[iidx.at[p * nch + j]],
                    ipairs.at[pl.ds(j * CHUNK, CHUNK)], sem))
            for c in copies:
                c.wait()
            pltpu.sync_copy(upairs, uout_hbm.at[pl.ds(base + p * half, half)])
            pltpu.sync_copy(ipairs, iout_hbm.at[pl.ds(base + p * half, half)])

    return gather_k, nw, bpw, nch


def _mlp_body(u_ref, i_ref, pu_ref, pi_ref, w0u_ref, w0i_ref, b0_ref,
              w1_ref, b1_ref, w2_ref, b2_ref, wo_ref, bo_ref, o_ref):
    f32 = jnp.float32
    he = jnp.dot(u_ref[:, :EMBED], w0u_ref[...], preferred_element_type=f32)
    ho = jnp.dot(u_ref[:, EMBED:], w0u_ref[...], preferred_element_type=f32)
    h = he + pu_ref[...] * (ho - he)
    he = jnp.dot(i_ref[:, :EMBED], w0i_ref[...], preferred_element_type=f32)
    ho = jnp.dot(i_ref[:, EMBED:], w0i_ref[...], preferred_element_type=f32)
    h = h + he + pi_ref[...] * (ho - he)
    h = jnp.maximum(h + b0_ref[...], 0.0)
    h = jnp.dot(h, w1_ref[...], preferred_element_type=f32) + b1_ref[...]
    h = jnp.maximum(h, 0.0)
    h = jnp.dot(h, w2_ref[...], preferred_element_type=f32) + b2_ref[...]
    h = jnp.maximum(h, 0.0)
    z = jnp.dot(h, wo_ref[...], preferred_element_type=f32) + bo_ref[...]
    o_ref[...] = 1.0 / (1.0 + jnp.exp(-z))


def _mlp(u, i, pu, pi, W0, b0, W1, b1, W2, b2, Wo, bo,
         block_m=2048, interpret=False):
    w0u = W0.T[:EMBED]
    w0i = W0.T[EMBED:]
    w1t, w2t, wot = W1.T, W2.T, Wo.T
    b0r, b1r, b2r, bor = b0[None, :], b1[None, :], b2[None, :], bo[None, :]
    grid = (BATCH // block_m,)
    full = lambda m: (0, 0)
    row = lambda m: (m, 0)
    return pl.pallas_call(
        _mlp_body,
        grid=grid,
        in_specs=[
            pl.BlockSpec((block_m, PAIR), row),
            pl.BlockSpec((block_m, PAIR), row),
            pl.BlockSpec((block_m, 1), row),
            pl.BlockSpec((block_m, 1), row),
            pl.BlockSpec(w0u.shape, full),
            pl.BlockSpec(w0i.shape, full),
            pl.BlockSpec(b0r.shape, full),
            pl.BlockSpec(w1t.shape, full),
            pl.BlockSpec(b1r.shape, full),
            pl.BlockSpec(w2t.shape, full),
            pl.BlockSpec(b2r.shape, full),
            pl.BlockSpec(wot.shape, full),
            pl.BlockSpec(bor.shape, full),
        ],
        out_specs=pl.BlockSpec((block_m, 1), row),
        out_shape=jax.ShapeDtypeStruct((BATCH, 1), jnp.float32),
        compiler_params=pltpu.CompilerParams(
            dimension_semantics=("arbitrary",)),
        interpret=interpret,
    )(u, i, pu, pi, w0u, w0i, b0r, w1t, b1r, w2t, b2r, wot, bor)


def kernel(user_ids, item_ids, user_table, item_table,
           W0, b0, W1, b1, W2, b2, Wo, bo):
    nu, ni = user_table.shape[0] // 2, item_table.shape[0] // 2
    gather_k, nw, bpw, nch = _make_gather(nu, ni)
    uids = user_ids.astype(jnp.int32)
    iids = item_ids.astype(jnp.int32)
    uid2 = (uids >> 1).reshape(nw, 2 * nch, CHUNK)
    iid2 = (iids >> 1).reshape(nw, 2 * nch, CHUNK)
    utab_p = user_table.reshape(nu, PAIR)
    itab_p = item_table.reshape(ni, PAIR)
    u_pairs, i_pairs = gather_k(uid2, iid2, utab_p, itab_p)
    pu = (uids & 1).astype(jnp.float32)[:, None]
    pi = (iids & 1).astype(jnp.float32)[:, None]
    return _mlp(u_pairs, i_pairs, pu, pi, W0, b0, W1, b1, W2, b2, Wo, bo)


# speedup vs baseline: 1.1943x; 1.1943x over previous
import functools

import jax
import jax.numpy as jnp
from jax import lax
from jax.experimental import pallas as pl
from jax.experimental.pallas import tpu as pltpu
from jax.experimental.pallas import tpu_sc as plsc

BATCH = 16384
EMBED = 64
WIN = 1024
SLOTS = 3
CAP = 672
SENT = 1 << 22
TRASH = BATCH
OUTROWS = BATCH + 2048


def _scalar(v, l):
    return lax.squeeze(lax.slice(v, (l,), (l + 1,)), (0,))


@functools.lru_cache(maxsize=None)
def _make_gather(num_rows):
    info = plsc.get_sparse_core_info()
    nc, ns = info.num_cores, info.num_subcores
    nw = nc * ns
    rng = num_rows // nw
    nwin = (rng + 127 + WIN - 1) // WIN + 1
    ws_max = ((num_rows - WIN) // 128) * 128
    tail_ws = (num_rows // 128) * 128

    mesh = plsc.VectorSubcoreMesh(core_axis_name="c", subcore_axis_name="s")

    @functools.partial(
        pl.kernel,
        mesh=mesh,
        out_type=[
            jax.ShapeDtypeStruct((OUTROWS, EMBED), jnp.float32),
            jax.ShapeDtypeStruct((OUTROWS, EMBED), jnp.float32),
        ],
        scratch_types=[
            pltpu.VMEM((CAP + 16,), jnp.int32),
            pltpu.VMEM((CAP + 16,), jnp.int32),
            pltpu.VMEM((EMBED, WIN + 128), jnp.float32),
            pltpu.VMEM((EMBED, CAP + 16), jnp.float32),
            pltpu.VMEM((32, EMBED), jnp.float32),
            pltpu.SemaphoreType.DMA,
        ],
        compiler_params=pltpu.CompilerParams(needs_layout_passes=False),
    )
    def gather_k(uid_hbm, upos_hbm, iid_hbm, ipos_hbm, utabT_hbm, itabT_hbm,
                 uout_hbm, iout_hbm, ids_v, pos_v, win, outbufT, stage, sem):
        wid = lax.axis_index("s") * nc + lax.axis_index("c")
        lanes = lax.iota(jnp.int32, 16)
        astart = ((wid * rng) >> 7) << 7

        def run_table(id_hbm, p_hbm, tabT_hbm, out_hbm):
            pltpu.sync_copy(id_hbm.at[wid], ids_v)
            pltpu.sync_copy(p_hbm.at[wid], pos_v)
            toff = wid * 0 + tail_ws
            pltpu.sync_copy(
                tabT_hbm.at[:, pl.ds(pl.multiple_of(toff, 128), 128)],
                win.at[:, pl.ds(WIN, 128)])

            def gstep(cur_v, ws):
                cur = _scalar(cur_v, 0)
                idv = ids_v[pl.ds(cur, 16)]
                rel = idv - ws
                tail_l = idv >= tail_ws
                ok = jnp.logical_or(
                    jnp.logical_and(rel < WIN, jnp.logical_not(tail_l)),
                    jnp.logical_and(tail_l, idv < num_rows))
                oki = ok.astype(jnp.int32)
                pfx = jnp.cumsum(1 - oki) == 0
                okp = jnp.logical_and(ok, pfx)
                h_v = plsc.all_reduce_population_count(okp)
                r = jnp.where(tail_l, WIN + (idv - tail_ws), rel)
                rc = jnp.clip(r, 0, WIN + 127)
                for l in range(16):
                    rl = _scalar(rc, l)
                    col = jnp.zeros((16,), jnp.int32) + (cur + l)
                    for k4 in range(4):
                        vals = plsc.load_gather(
                            win, [lanes + 16 * k4,
                                  jnp.zeros((16,), jnp.int32) + rl])
                        plsc.store_scatter(outbufT, [lanes + 16 * k4, col],
                                           vals)
                return cur_v + h_v

            def wbody(k, cur_v):
                ws = jnp.minimum(astart + WIN * k, ws_max)
                pltpu.sync_copy(
                    tabT_hbm.at[:, pl.ds(pl.multiple_of(ws, 128), WIN)],
                    win.at[:, pl.ds(0, WIN)])
                for _ in range(SLOTS):
                    cur_v = gstep(cur_v, ws)
                return cur_v

            lax.fori_loop(0, nwin, wbody, jnp.zeros((16,), jnp.int32),
                          unroll=False)

            def fin(t, c):
                j0 = t * 32
                for q in range(32):
                    cq = jnp.zeros((16,), jnp.int32) + (j0 + q)
                    for k4 in range(4):
                        vals = plsc.load_gather(
                            outbufT, [lanes + 16 * k4, cq])
                        stage[q, pl.ds(16 * k4, 16)] = vals
                posv0 = pos_v[pl.ds(j0, 16)]
                posv1 = pos_v[pl.ds(j0 + 16, 16)]
                for q in range(16):
                    pltpu.async_copy(stage.at[q],
                                     out_hbm.at[_scalar(posv0, q)], sem)
                    pltpu.async_copy(stage.at[16 + q],
                                     out_hbm.at[_scalar(posv1, q)], sem)
                pltpu.make_async_copy(out_hbm.at[pl.ds(0, 32)], stage,
                                      sem).wait()
                return c

            lax.fori_loop(0, CAP // 32, fin, 0, unroll=False)

        run_table(uid_hbm, upos_hbm, utabT_hbm, uout_hbm)
        run_table(iid_hbm, ipos_hbm, itabT_hbm, iout_hbm)

    return gather_k, nw, rng


def _mlp_body(u_ref, i_ref, w0u_ref, w0i_ref, b0_ref, w1_ref, b1_ref,
              w2_ref, b2_ref, wo_ref, bo_ref, o_ref):
    h = jnp.dot(u_ref[...], w0u_ref[...], preferred_element_type=jnp.float32)
    h = h + jnp.dot(i_ref[...], w0i_ref[...], preferred_element_type=jnp.float32)
    h = jnp.maximum(h + b0_ref[...], 0.0)
    h = jnp.dot(h, w1_ref[...], preferred_element_type=jnp.float32) + b1_ref[...]
    h = jnp.maximum(h, 0.0)
    h = jnp.dot(h, w2_ref[...], preferred_element_type=jnp.float32) + b2_ref[...]
    h = jnp.maximum(h, 0.0)
    z = jnp.dot(h, wo_ref[...], preferred_element_type=jnp.float32) + bo_ref[...]
    o_ref[...] = 1.0 / (1.0 + jnp.exp(-z))


def _mlp(u, i, W0, b0, W1, b1, W2, b2, Wo, bo, block_m=2048, interpret=False):
    w0u = W0.T[:EMBED]
    w0i = W0.T[EMBED:]
    w1t, w2t, wot = W1.T, W2.T, Wo.T
    b0r, b1r, b2r, bor = b0[None, :], b1[None, :], b2[None, :], bo[None, :]
    grid = (BATCH // block_m,)
    full = lambda m: (0, 0)
    return pl.pallas_call(
        _mlp_body,
        grid=grid,
        in_specs=[
            pl.BlockSpec((block_m, EMBED), lambda m: (m, 0)),
            pl.BlockSpec((block_m, EMBED), lambda m: (m, 0)),
            pl.BlockSpec(w0u.shape, full),
            pl.BlockSpec(w0i.shape, full),
            pl.BlockSpec(b0r.shape, full),
            pl.BlockSpec(w1t.shape, full),
            pl.BlockSpec(b1r.shape, full),
            pl.BlockSpec(w2t.shape, full),
            pl.BlockSpec(b2r.shape, full),
            pl.BlockSpec(wot.shape, full),
            pl.BlockSpec(bor.shape, full),
        ],
        out_specs=pl.BlockSpec((block_m, 1), lambda m: (m, 0)),
        out_shape=jax.ShapeDtypeStruct((BATCH, 1), jnp.float32),
        compiler_params=pltpu.CompilerParams(
            dimension_semantics=("arbitrary",)),
        interpret=interpret,
    )(u, i, w0u, w0i, b0r, w1t, b1r, w2t, b2r, wot, bor)


def _partition(ids, nw, rng):
    pos = lax.iota(jnp.int32, BATCH)
    sid, spos = lax.sort([ids, pos], num_keys=1)
    bounds = jnp.arange(nw + 1, dtype=jnp.int32) * rng
    start = jnp.searchsorted(sid, bounds[:-1]).astype(jnp.int32)
    end = jnp.searchsorted(sid, bounds[1:]).astype(jnp.int32)
    idx = start[:, None] + jnp.arange(CAP + 16, dtype=jnp.int32)[None, :]
    valid = idx < end[:, None]
    idxc = jnp.minimum(idx, BATCH - 1)
    ids2 = jnp.where(valid, sid[idxc], SENT)
    pos2 = jnp.where(valid, spos[idxc], TRASH)
    return ids2, pos2


def kernel(user_ids, item_ids, user_table, item_table,
           W0, b0, W1, b1, W2, b2, Wo, bo):
    gather_k, nw, rng = _make_gather(user_table.shape[0])
    uid2, upos2 = _partition(user_ids.astype(jnp.int32), nw, rng)
    iid2, ipos2 = _partition(item_ids.astype(jnp.int32), nw, rng)
    u_rows, i_rows = gather_k(uid2, upos2, iid2, ipos2,
                              user_table.T, item_table.T)
    return _mlp(u_rows, i_rows, W0, b0, W1, b1, W2, b2, Wo, bo)
